# Initial kernel scaffold; baseline (speedup 1.0000x reference)
import jax
import jax.numpy as jnp
from jax.experimental import pallas as pl


def kernel(dense_x, lS_i, emb, bot_Ws, bot_bs, top_Ws, top_bs):
    raise NotImplementedError("write your pallas kernel here")


# speedup vs baseline: 1.1958x; 1.1958x over previous
import functools

import numpy as np
import jax
import jax.numpy as jnp
from jax import lax
from jax.experimental import pallas as pl
from jax.experimental.pallas import tpu as pltpu
from jax.experimental.pallas import tpu_sc as plsc

NUM_TABLES = 26
VOCAB = 100000
DIM = 64
HIST = 200
HIST_PAD = 208
CHUNK = 104
LANES = 16

_NI = NUM_TABLES + 1
_PAIRS = [(i, j) for i in range(_NI) for j in range(i)]
NPAIR = len(_PAIRS)

_SX = np.zeros((NPAIR, NUM_TABLES), np.float32)
_A2 = np.zeros((NPAIR, NUM_TABLES), np.float32)
_B2T = np.zeros((NPAIR, NUM_TABLES), np.float32)
for _p, (_i, _j) in enumerate(_PAIRS):
    if _j == 0:
        _SX[_p, _i - 1] = 1.0
    else:
        _A2[_p, _i - 1] = 1.0
        _B2T[_p, _j - 1] = 1.0


def _bags_body(idx_hbm, table_hbm, out_hbm, idx_v, rows_v, ly_v, sem):
    w = lax.axis_index("s") * 2 + lax.axis_index("c")

    @pl.when(w < NUM_TABLES)
    def _():
        pltpu.sync_copy(idx_hbm.at[w], idx_v)
        off = w * VOCAB
        for j in range(HIST_PAD // LANES):
            sl = pl.ds(j * LANES, LANES)
            idx_v[sl] = idx_v[sl] + off
        c0 = pltpu.async_copy(
            table_hbm.at[idx_v.at[pl.ds(0, CHUNK)]], rows_v.at[pl.ds(0, CHUNK)], sem)
        c1 = pltpu.async_copy(
            table_hbm.at[idx_v.at[pl.ds(CHUNK, CHUNK)]], rows_v.at[pl.ds(CHUNK, CHUNK)], sem)
        c0.wait()
        c1.wait()

        def body(i, acc):
            return tuple(
                acc[j] + rows_v[i, pl.ds(j * LANES, LANES)] for j in range(DIM // LANES))

        acc = lax.fori_loop(
            0, HIST, body,
            tuple(jnp.zeros((LANES,), jnp.float32) for _ in range(DIM // LANES)))
        for j in range(DIM // LANES):
            ly_v[pl.ds(j * LANES, LANES)] = acc[j]
        pltpu.sync_copy(ly_v, out_hbm.at[w])


def _bags(idx_pad, emb_flat):
    mesh = plsc.VectorSubcoreMesh(core_axis_name="c", subcore_axis_name="s")
    f = functools.partial(
        pl.kernel,
        out_type=jax.ShapeDtypeStruct((NUM_TABLES, DIM), jnp.float32),
        mesh=mesh,
        scratch_types=[
            pltpu.VMEM((HIST_PAD,), jnp.int32),
            pltpu.VMEM((HIST_PAD, DIM), jnp.float32),
            pltpu.VMEM((DIM,), jnp.float32),
            pltpu.SemaphoreType.DMA,
        ],
        compiler_params=pltpu.CompilerParams(use_tc_tiling_on_sc=False),
    )(_bags_body)
    return f(idx_pad, emb_flat)


def _mv(W, x):
    return lax.dot_general(W, x, (((1,), (0,)), ((), ())),
                           preferred_element_type=jnp.float32)


def _mlp_body(x_ref, l_ref, sx_ref, a2_ref, b2t_ref,
              bw0, bb0, bw1, bb1, bw2, bb2,
              tw0x, tw0z, tb0, tw1, tb1, tw2, tb2, out_ref):
    x = x_ref[...]
    x = jnp.maximum(_mv(bw0[...], x) + bb0[...], 0.0)
    x = jnp.maximum(_mv(bw1[...], x) + bb1[...], 0.0)
    x = jnp.maximum(_mv(bw2[...], x) + bb2[...], 0.0)

    L = l_ref[...]
    zx = _mv(L, x)
    ZL = lax.dot_general(L, L, (((1,), (1,)), ((), ())),
                         preferred_element_type=jnp.float32)
    zf = _mv(sx_ref[...], zx)
    ZA = lax.dot_general(a2_ref[...], ZL, (((1,), (0,)), ((), ())),
                         preferred_element_type=jnp.float32)
    zf = zf + jnp.sum(ZA * b2t_ref[...], axis=1, keepdims=True)

    h = jnp.maximum(_mv(tw0x[...], x) + _mv(tw0z[...], zf) + tb0[...], 0.0)
    h = jnp.maximum(_mv(tw1[...], h) + tb1[...], 0.0)
    h = _mv(tw2[...], h) + tb2[...]
    out_ref[...] = jax.nn.sigmoid(h)


def kernel(dense_x, lS_i, emb, bot_Ws, bot_bs, top_Ws, top_bs):
    idx = lS_i[:, 0, :]
    idx_pad = jnp.pad(idx, ((0, 0), (0, HIST_PAD - HIST)))
    emb_flat = emb.reshape(NUM_TABLES * VOCAB, DIM)
    ly = _bags(idx_pad, emb_flat)

    x0 = dense_x.reshape(13, 1)
    sx = jnp.asarray(_SX)
    a2 = jnp.asarray(_A2)
    b2t = jnp.asarray(_B2T)
    bb = [b.reshape(-1, 1) for b in bot_bs]
    tb = [b.reshape(-1, 1) for b in top_bs]
    tw0x = top_Ws[0][:, :DIM]
    tw0z = top_Ws[0][:, DIM:]

    return pl.pallas_call(
        _mlp_body,
        out_shape=jax.ShapeDtypeStruct((1, 1), jnp.float32),
    )(x0, ly, sx, a2, b2t,
      bot_Ws[0], bb[0], bot_Ws[1], bb[1], bot_Ws[2], bb[2],
      tw0x, tw0z, tb[0], top_Ws[1], tb[1], top_Ws[2], tb[2])


# speedup vs baseline: 1.9750x; 1.6516x over previous
import functools

import numpy as np
import jax
import jax.numpy as jnp
from jax import lax
from jax.experimental import pallas as pl
from jax.experimental.pallas import tpu as pltpu
from jax.experimental.pallas import tpu_sc as plsc

NUM_TABLES = 26
VOCAB = 100000
DIM = 64
HIST = 200
HIST_PAD = 208
LANES = 16

_NI = NUM_TABLES + 1
_PAIRS = [(i, j) for i in range(_NI) for j in range(i)]
NPAIR = len(_PAIRS)

_SX = np.zeros((NPAIR, NUM_TABLES), np.float32)
_A2 = np.zeros((NPAIR, NUM_TABLES), np.float32)
_B2T = np.zeros((NPAIR, NUM_TABLES), np.float32)
for _p, (_i, _j) in enumerate(_PAIRS):
    if _j == 0:
        _SX[_p, _i - 1] = 1.0
    else:
        _A2[_p, _i - 1] = 1.0
        _B2T[_p, _j - 1] = 1.0


def _bags_body(idx_hbm, table_hbm, out_hbm, idx_v, rows_v, ly_v, sem):
    w = lax.axis_index("s") * 2 + lax.axis_index("c")

    @pl.when(w < NUM_TABLES)
    def _():
        pltpu.sync_copy(idx_hbm.at[w], idx_v)

        def issue(g, carry):
            base = g * LANES
            vv = idx_v[pl.ds(base, LANES)]
            for j in range(LANES):
                pltpu.make_async_copy(
                    table_hbm.at[w, vv[j]], rows_v.at[base + j], sem).start()
            return carry

        lax.fori_loop(0, HIST_PAD // LANES, issue, 0)
        pltpu.make_async_copy(
            table_hbm.at[w, pl.ds(0, HIST_PAD)], rows_v.at[pl.ds(0, HIST_PAD)],
            sem).wait()

        def body(i, acc):
            return tuple(
                acc[j] + rows_v[i, pl.ds(j * LANES, LANES)] for j in range(DIM // LANES))

        acc = lax.fori_loop(
            0, HIST, body,
            tuple(jnp.zeros((LANES,), jnp.float32) for _ in range(DIM // LANES)))
        for j in range(DIM // LANES):
            ly_v[pl.ds(j * LANES, LANES)] = acc[j]
        pltpu.sync_copy(ly_v, out_hbm.at[w])


def _bags(idx, emb):
    mesh = plsc.VectorSubcoreMesh(core_axis_name="c", subcore_axis_name="s")
    f = functools.partial(
        pl.kernel,
        out_type=jax.ShapeDtypeStruct((NUM_TABLES, DIM), jnp.float32),
        mesh=mesh,
        scratch_types=[
            pltpu.VMEM((HIST_PAD,), jnp.int32),
            pltpu.VMEM((HIST_PAD, DIM), jnp.float32),
            pltpu.VMEM((DIM,), jnp.float32),
            pltpu.SemaphoreType.DMA,
        ],
    )(_bags_body)
    return f(idx, emb)


def _mv(W, x):
    return lax.dot_general(W, x, (((1,), (0,)), ((), ())),
                           preferred_element_type=jnp.float32)


def _mlp_body(x_ref, l_ref, sx_ref, a2_ref, b2t_ref,
              bw0, bb0, bw1, bb1, bw2, bb2,
              tw0x, tw0z, tb0, tw1, tb1, tw2, tb2, out_ref):
    x = x_ref[...]
    x = jnp.maximum(_mv(bw0[...], x) + bb0[...], 0.0)
    x = jnp.maximum(_mv(bw1[...], x) + bb1[...], 0.0)
    x = jnp.maximum(_mv(bw2[...], x) + bb2[...], 0.0)

    L = l_ref[...]
    zx = _mv(L, x)
    ZL = lax.dot_general(L, L, (((1,), (1,)), ((), ())),
                         preferred_element_type=jnp.float32)
    zf = _mv(sx_ref[...], zx)
    ZA = lax.dot_general(a2_ref[...], ZL, (((1,), (0,)), ((), ())),
                         preferred_element_type=jnp.float32)
    zf = zf + jnp.sum(ZA * b2t_ref[...], axis=1, keepdims=True)

    h = jnp.maximum(_mv(tw0x[...], x) + _mv(tw0z[...], zf) + tb0[...], 0.0)
    h = jnp.maximum(_mv(tw1[...], h) + tb1[...], 0.0)
    h = _mv(tw2[...], h) + tb2[...]
    out_ref[...] = jax.nn.sigmoid(h)


def kernel(dense_x, lS_i, emb, bot_Ws, bot_bs, top_Ws, top_bs):
    idx = jnp.pad(lS_i[:, 0, :], ((0, 0), (0, HIST_PAD - HIST)))
    ly = _bags(idx, emb)

    x0 = dense_x.reshape(13, 1)
    sx = jnp.asarray(_SX)
    a2 = jnp.asarray(_A2)
    b2t = jnp.asarray(_B2T)
    bb = [b.reshape(-1, 1) for b in bot_bs]
    tb = [b.reshape(-1, 1) for b in top_bs]
    tw0x = top_Ws[0][:, :DIM]
    tw0z = top_Ws[0][:, DIM:]

    return pl.pallas_call(
        _mlp_body,
        out_shape=jax.ShapeDtypeStruct((1, 1), jnp.float32),
    )(x0, ly, sx, a2, b2t,
      bot_Ws[0], bb[0], bot_Ws[1], bb[1], bot_Ws[2], bb[2],
      tw0x, tw0z, tb[0], top_Ws[1], tb[1], top_Ws[2], tb[2])


# speedup vs baseline: 15.9921x; 8.0975x over previous
import functools

import numpy as np
import jax
import jax.numpy as jnp
from jax import lax
from jax.experimental import pallas as pl
from jax.experimental.pallas import tpu as pltpu
from jax.experimental.pallas import tpu_sc as plsc

NUM_TABLES = 26
VOCAB = 100000
DIM = 64
HIST = 200
HIST_PAD = 208
LANES = 16

_NI = NUM_TABLES + 1
_PAIRS = [(i, j) for i in range(_NI) for j in range(i)]
NPAIR = len(_PAIRS)

_SX = np.zeros((NPAIR, NUM_TABLES), np.float32)
_A2 = np.zeros((NPAIR, NUM_TABLES), np.float32)
_B2T = np.zeros((NPAIR, NUM_TABLES), np.float32)
for _p, (_i, _j) in enumerate(_PAIRS):
    if _j == 0:
        _SX[_p, _i - 1] = 1.0
    else:
        _A2[_p, _i - 1] = 1.0
        _B2T[_p, _j - 1] = 1.0


GRP = 8
WIN = 128
_NGRP = HIST_PAD // GRP

def _bags_body(idx_hbm, table_hbm, out_hbm, idx_v, blk, ly_v, sem):
    w = lax.axis_index("s") * 2 + lax.axis_index("c")

    @pl.when(w < NUM_TABLES)
    def _():
        pltpu.sync_copy(idx_hbm.at[w], idx_v)
        iota16 = lax.iota(jnp.int32, LANES)

        def group(g, acc):
            base = pl.multiple_of(g * GRP, GRP)
            vv = idx_v[pl.ds(base, LANES)]
            offs = []
            for s in range(GRP):
                off = pl.multiple_of((vv[s] // WIN) * WIN, WIN)
                offs.append(off)
                pltpu.make_async_copy(
                    table_hbm.at[w, :, pl.ds(off, WIN)], blk.at[s], sem).start()
            for s in range(GRP):
                pltpu.make_async_copy(
                    table_hbm.at[w, :, pl.ds(0, WIN)], blk.at[s], sem).wait()
            for s in range(GRP):
                lane = jnp.full((LANES,), vv[s] - offs[s], jnp.int32)
                slot = jnp.full((LANES,), s, jnp.int32)
                acc = tuple(
                    acc[j] + plsc.load_gather(
                        blk, [slot, iota16 + 16 * j, lane])
                    for j in range(DIM // LANES))
            return acc

        acc = lax.fori_loop(
            0, HIST // GRP, group,
            tuple(jnp.zeros((LANES,), jnp.float32) for _ in range(DIM // LANES)))
        for j in range(DIM // LANES):
            ly_v[pl.ds(j * LANES, LANES)] = acc[j]
        pltpu.sync_copy(ly_v, out_hbm.at[w])


def _bags(idx, emb):
    mesh = plsc.VectorSubcoreMesh(core_axis_name="c", subcore_axis_name="s")
    f = functools.partial(
        pl.kernel,
        out_type=jax.ShapeDtypeStruct((NUM_TABLES, DIM), jnp.float32),
        mesh=mesh,
        scratch_types=[
            pltpu.VMEM((HIST_PAD,), jnp.int32),
            pltpu.VMEM((GRP, DIM, WIN), jnp.float32),
            pltpu.VMEM((DIM,), jnp.float32),
            pltpu.SemaphoreType.DMA,
        ],
        compiler_params=pltpu.CompilerParams(needs_layout_passes=False),
    )(_bags_body)
    return f(idx, emb)


def _mv(W, x):
    return lax.dot_general(W, x, (((1,), (0,)), ((), ())),
                           preferred_element_type=jnp.float32)


def _mlp_body(x_ref, l_ref, sx_ref, a2_ref, b2t_ref,
              bw0, bb0, bw1, bb1, bw2, bb2,
              tw0x, tw0z, tb0, tw1, tb1, tw2, tb2, out_ref):
    x = x_ref[...]
    x = jnp.maximum(_mv(bw0[...], x) + bb0[...], 0.0)
    x = jnp.maximum(_mv(bw1[...], x) + bb1[...], 0.0)
    x = jnp.maximum(_mv(bw2[...], x) + bb2[...], 0.0)

    L = l_ref[...]
    zx = _mv(L, x)
    ZL = lax.dot_general(L, L, (((1,), (1,)), ((), ())),
                         preferred_element_type=jnp.float32)
    zf = _mv(sx_ref[...], zx)
    ZA = lax.dot_general(a2_ref[...], ZL, (((1,), (0,)), ((), ())),
                         preferred_element_type=jnp.float32)
    zf = zf + jnp.sum(ZA * b2t_ref[...], axis=1, keepdims=True)

    h = jnp.maximum(_mv(tw0x[...], x) + _mv(tw0z[...], zf) + tb0[...], 0.0)
    h = jnp.maximum(_mv(tw1[...], h) + tb1[...], 0.0)
    h = _mv(tw2[...], h) + tb2[...]
    out_ref[...] = jax.nn.sigmoid(h)


def kernel(dense_x, lS_i, emb, bot_Ws, bot_bs, top_Ws, top_bs):
    idx = jnp.pad(lS_i[:, 0, :], ((0, 0), (0, HIST_PAD - HIST)))
    ly = _bags(idx, emb.transpose(0, 2, 1))

    x0 = dense_x.reshape(13, 1)
    sx = jnp.asarray(_SX)
    a2 = jnp.asarray(_A2)
    b2t = jnp.asarray(_B2T)
    bb = [b.reshape(-1, 1) for b in bot_bs]
    tb = [b.reshape(-1, 1) for b in top_bs]
    tw0x = top_Ws[0][:, :DIM]
    tw0z = top_Ws[0][:, DIM:]

    return pl.pallas_call(
        _mlp_body,
        out_shape=jax.ShapeDtypeStruct((1, 1), jnp.float32),
    )(x0, ly, sx, a2, b2t,
      bot_Ws[0], bb[0], bot_Ws[1], bb[1], bot_Ws[2], bb[2],
      tw0x, tw0z, tb[0], top_Ws[1], tb[1], top_Ws[2], tb[2])


# speedup vs baseline: 17.8012x; 1.1131x over previous
import functools

import numpy as np
import jax
import jax.numpy as jnp
from jax import lax
from jax.experimental import pallas as pl
from jax.experimental.pallas import tpu as pltpu
from jax.experimental.pallas import tpu_sc as plsc

NUM_TABLES = 26
VOCAB = 100000
DIM = 64
HIST = 200
LANES = 16
NTILES = 32
NLOOK = NUM_TABLES * HIST
PER_TILE = 168
GRP = 8
WIN = 128
NGRP = PER_TILE // GRP
IDX_PAD = PER_TILE * NTILES + LANES

_NI = NUM_TABLES + 1
_PAIRS = [(i, j) for i in range(_NI) for j in range(i)]
NPAIR = len(_PAIRS)

_SX = np.zeros((NPAIR, NUM_TABLES), np.float32)
_A2 = np.zeros((NPAIR, NUM_TABLES), np.float32)
_B2T = np.zeros((NPAIR, NUM_TABLES), np.float32)
for _p, (_i, _j) in enumerate(_PAIRS):
    if _j == 0:
        _SX[_p, _i - 1] = 1.0
    else:
        _A2[_p, _i - 1] = 1.0
        _B2T[_p, _j - 1] = 1.0

_COMB = np.zeros((NUM_TABLES, 2 * NTILES), np.float32)
for _t in range(NTILES):
    if PER_TILE * _t >= NLOOK:
        continue
    _k0 = min((PER_TILE * _t) // HIST, NUM_TABLES - 1)
    _COMB[_k0, 2 * _t] = 1.0
    _COMB[min(_k0 + 1, NUM_TABLES - 1), 2 * _t + 1] = 1.0


def _bags_body(idx_hbm, table_hbm, out_hbm, idx_v, blk, ly_v, sem):
    t = lax.axis_index("s") * 2 + lax.axis_index("c")
    q0 = t * PER_TILE

    @pl.when(q0 < NLOOK)
    def _():
        pltpu.sync_copy(
            idx_hbm.at[pl.ds(pl.multiple_of(q0, 8), PER_TILE + LANES)], idx_v)
        iota16 = lax.iota(jnp.int32, LANES)
        k0 = jnp.minimum(q0 // HIST, NUM_TABLES - 1)

        def group(g, acc):
            base = pl.multiple_of(g * GRP, GRP)
            vv = idx_v[pl.ds(base, LANES)]
            lanes = []
            for s in range(GRP):
                off = pl.multiple_of((vv[s] // WIN) * WIN, WIN)
                q = q0 + base + s
                k = jnp.minimum(q // HIST, NUM_TABLES - 1)
                lanes.append((vv[s] - off, k - k0, q < NLOOK))
                pltpu.make_async_copy(
                    table_hbm.at[k, :, pl.ds(off, WIN)], blk.at[s], sem).start()
            for s in range(GRP):
                pltpu.make_async_copy(
                    table_hbm.at[k0, :, pl.ds(0, WIN)], blk.at[s], sem).wait()
            for s in range(GRP):
                lane, slot, live = lanes[s]
                w0 = jnp.where(live & (slot == 0), 1.0, 0.0)
                w1 = jnp.where(live & (slot == 1), 1.0, 0.0)
                w0v = jnp.full((LANES,), w0, jnp.float32)
                w1v = jnp.full((LANES,), w1, jnp.float32)
                lanev = jnp.full((LANES,), lane, jnp.int32)
                slotv = jnp.full((LANES,), s, jnp.int32)
                new = []
                for j in range(DIM // LANES):
                    g_ = plsc.load_gather(blk, [slotv, iota16 + 16 * j, lanev])
                    new.append(acc[j] + g_ * w0v)
                    new.append(acc[DIM // LANES + j] + g_ * w1v)
                acc = tuple(new[::2]) + tuple(new[1::2])
            return acc

        acc = lax.fori_loop(
            0, NGRP, group,
            tuple(jnp.zeros((LANES,), jnp.float32)
                  for _ in range(2 * (DIM // LANES))))
        for u in range(2):
            for j in range(DIM // LANES):
                ly_v[u, pl.ds(j * LANES, LANES)] = acc[u * (DIM // LANES) + j]
        pltpu.sync_copy(ly_v, out_hbm.at[t])


def _bags(idx_flat, emb):
    mesh = plsc.VectorSubcoreMesh(core_axis_name="c", subcore_axis_name="s")
    f = functools.partial(
        pl.kernel,
        out_type=jax.ShapeDtypeStruct((NTILES, 2, DIM), jnp.float32),
        mesh=mesh,
        scratch_types=[
            pltpu.VMEM((PER_TILE + LANES,), jnp.int32),
            pltpu.VMEM((GRP, DIM, WIN), jnp.float32),
            pltpu.VMEM((2, DIM), jnp.float32),
            pltpu.SemaphoreType.DMA,
        ],
        compiler_params=pltpu.CompilerParams(needs_layout_passes=False),
    )(_bags_body)
    return f(idx_flat, emb)


def _mv(W, x):
    return lax.dot_general(W, x, (((1,), (0,)), ((), ())),
                           preferred_element_type=jnp.float32)


def _mlp_body(x_ref, parts_ref, comb_ref, sx_ref, a2_ref, b2t_ref,
              bw0, bb0, bw1, bb1, bw2, bb2,
              tw0x, tw0z, tb0, tw1, tb1, tw2, tb2, out_ref):
    x = x_ref[...]
    x = jnp.maximum(_mv(bw0[...], x) + bb0[...], 0.0)
    x = jnp.maximum(_mv(bw1[...], x) + bb1[...], 0.0)
    x = jnp.maximum(_mv(bw2[...], x) + bb2[...], 0.0)

    L = lax.dot_general(comb_ref[...], parts_ref[...], (((1,), (0,)), ((), ())),
                        preferred_element_type=jnp.float32)
    zx = _mv(L, x)
    ZL = lax.dot_general(L, L, (((1,), (1,)), ((), ())),
                         preferred_element_type=jnp.float32)
    zf = _mv(sx_ref[...], zx)
    ZA = lax.dot_general(a2_ref[...], ZL, (((1,), (0,)), ((), ())),
                         preferred_element_type=jnp.float32)
    zf = zf + jnp.sum(ZA * b2t_ref[...], axis=1, keepdims=True)

    h = jnp.maximum(_mv(tw0x[...], x) + _mv(tw0z[...], zf) + tb0[...], 0.0)
    h = jnp.maximum(_mv(tw1[...], h) + tb1[...], 0.0)
    h = _mv(tw2[...], h) + tb2[...]
    out_ref[...] = jax.nn.sigmoid(h)


def kernel(dense_x, lS_i, emb, bot_Ws, bot_bs, top_Ws, top_bs):
    idx_flat = jnp.pad(lS_i[:, 0, :].reshape(-1), (0, IDX_PAD - NLOOK))
    parts = _bags(idx_flat, emb.transpose(0, 2, 1))
    parts = parts.reshape(2 * NTILES, DIM)

    x0 = dense_x.reshape(13, 1)
    comb = jnp.asarray(_COMB)
    sx = jnp.asarray(_SX)
    a2 = jnp.asarray(_A2)
    b2t = jnp.asarray(_B2T)
    bb = [b.reshape(-1, 1) for b in bot_bs]
    tb = [b.reshape(-1, 1) for b in top_bs]
    tw0x = top_Ws[0][:, :DIM]
    tw0z = top_Ws[0][:, DIM:]

    return pl.pallas_call(
        _mlp_body,
        out_shape=jax.ShapeDtypeStruct((1, 1), jnp.float32),
    )(x0, parts, comb, sx, a2, b2t,
      bot_Ws[0], bb[0], bot_Ws[1], bb[1], bot_Ws[2], bb[2],
      tw0x, tw0z, tb[0], top_Ws[1], tb[1], top_Ws[2], tb[2])
